# baseline (speedup 1.0000x reference)
import functools

import jax
import jax.numpy as jnp
from jax.experimental import pallas as pl
from jax.experimental.pallas import tpu as pltpu

N = 10000
E = 320000
EPAD = 79 * 128 * 32
NPAD = 79 * 128
DUMP = N
EPS = 1e-5
BE = 2048
G = EPAD // BE


def _pad_e(a, fill):
    return jnp.concatenate([a, jnp.full((EPAD - E,), fill, a.dtype)])


def _mm_stats_body(d_ref, w_ref, gam_ref, del_ref, W_ref, v_ref, st_ref, acc_ref):
    i = pl.program_id(0)

    @pl.when(i == 0)
    def _init():
        acc_ref[...] = jnp.zeros_like(acc_ref)

    n = gam_ref[...] * d_ref[...] + del_ref[...]
    v = jnp.dot(jax.nn.relu(n), W_ref[...], preferred_element_type=jnp.float32)
    v_ref[...] = v
    w = (w_ref[...] < N).astype(jnp.float32)
    wv = v * w
    acc_ref[0:1, :] += jnp.sum(wv, axis=0, keepdims=True)
    acc_ref[1:2, :] += jnp.sum(wv * v, axis=0, keepdims=True)

    @pl.when(i == G - 1)
    def _fin():
        st_ref[...] = acc_ref[...]


@jax.jit
def _mm_stats(d, w_col, gamma, delta, W1bT):
    return pl.pallas_call(
        _mm_stats_body,
        grid=(G,),
        in_specs=[
            pl.BlockSpec((BE, 128), lambda i: (i, 0)),
            pl.BlockSpec((BE, 1), lambda i: (i, 0)),
            pl.BlockSpec((1, 128), lambda i: (0, 0)),
            pl.BlockSpec((1, 128), lambda i: (0, 0)),
            pl.BlockSpec((128, 128), lambda i: (0, 0)),
        ],
        out_specs=[
            pl.BlockSpec((BE, 128), lambda i: (i, 0)),
            pl.BlockSpec((8, 128), lambda i: (0, 0)),
        ],
        out_shape=[
            jax.ShapeDtypeStruct((EPAD, 128), jnp.float32),
            jax.ShapeDtypeStruct((8, 128), jnp.float32),
        ],
        scratch_shapes=[pltpu.VMEM((8, 128), jnp.float32)],
    )(d, w_col, gamma.reshape(1, 128), delta.reshape(1, 128), W1bT)


def _gather_diff_scatter(x, src, dst, dst_eff):
    xs = jnp.take(x, src, axis=0)
    d = xs - jnp.take(x, dst, axis=0)
    P = jnp.zeros((NPAD, 128), jnp.float32).at[dst_eff].add(xs)
    return d, P


def _scatter_rows(rows, dst_eff):
    return jnp.zeros((NPAD, 128), jnp.float32).at[dst_eff].add(rows)


def _final_edge(u, vmat, dst, dst_eff, A, B):
    z = jax.nn.relu(A * (jnp.take(u, dst, axis=0) + vmat) + B)
    return jnp.zeros((NPAD, 128), jnp.float32).at[dst_eff].add(z)


def _conv(x, src, dst, dst_eff, w_col, cnt_src, cnt_dst, Wsum, p):
    relu = jax.nn.relu
    d, P = _gather_diff_scatter(x, src, dst, dst_eff)
    cd = cnt_dst[:, None]
    cs = cnt_src[:, None]
    sx = (cd * x).sum(0)
    sx2 = (cd * x * x).sum(0)
    sj = (cs * x).sum(0)
    sj2 = (cs * x * x).sum(0)
    CT = (x * P[:N]).sum(0)
    m_xi = sx / Wsum
    v_xi = sx2 / Wsum - m_xi**2
    m_d = (sj - sx) / Wsum
    v_d = (sj2 + sx2 - 2 * CT) / Wsum - m_d**2
    alpha = p["g1"][:128] * jax.lax.rsqrt(v_xi + EPS)
    beta = p["b1"][:128] - m_xi * alpha
    gamma = p["g1"][128:] * jax.lax.rsqrt(v_d + EPS)
    delta = p["b1"][128:] - m_d * gamma
    u = relu(alpha * x + beta) @ p["W1"][:, :128].T
    vmat, st = _mm_stats(d, w_col, gamma, delta, p["W1"][:, 128:].T)
    Sv, Sv2 = st[0], st[1]
    Q = _scatter_rows(vmat, dst_eff)
    Sh = (cd * u).sum(0) + Sv
    Sh2 = (cd * u * u).sum(0) + 2 * (u * Q[:N]).sum(0) + Sv2
    m2 = Sh / Wsum
    var2 = Sh2 / Wsum - m2**2
    A = p["g2"] * jax.lax.rsqrt(var2 + EPS)
    B = p["b2"] - m2 * A
    S = _final_edge(u, vmat, dst, dst_eff, A, B)
    agg = S[:N] / jnp.maximum(cnt_dst, 1.0)[:, None]
    return agg @ p["W2"].T


def _bn_plain(x, g, b):
    m = x.mean(axis=0)
    v = x.var(axis=0)
    return (x - m) * jax.lax.rsqrt(v + EPS) * g + b


def kernel(x, edge_index, edge_delta, edge_self, audio_mask, params):
    relu = jax.nn.relu
    src = _pad_e(edge_index[0], 0)
    dst = _pad_e(edge_index[1], 0)
    m1 = _pad_e(edge_delta < 1, False)
    m2 = _pad_e(((edge_delta >= 1) & (edge_delta < 4)) | (edge_self == 1), False)
    subs = []
    for m in (m1, m2):
        dst_eff = jnp.where(m, dst, DUMP)
        src_eff = jnp.where(m, src, DUMP)
        cnt_dst = jnp.zeros((NPAD,), jnp.float32).at[dst_eff].add(1.0)[:N]
        cnt_src = jnp.zeros((NPAD,), jnp.float32).at[src_eff].add(1.0)[:N]
        Wsum = cnt_dst.sum()
        w_col = dst_eff[:, None]
        subs.append((dst_eff, w_col, cnt_src, cnt_dst, Wsum))
    p = params
    a = x[:, 0, :] @ p["W0a"].T + p["b0a"]
    v = x[:, 1, :] @ p["W0v"].T + p["b0v"]
    gf = jnp.where(audio_mask[:, None], a, v)
    gf = relu(_bn_plain(gf, p["g0"], p["b0"]))
    g = gf
    for li, (lp, gn, bn_) in enumerate([
        (p["l1"], p["gb1"], p["bb1"]),
        (p["l2"], p["gb2"], p["bb2"]),
        (p["l3"], p["gb3"], p["bb3"]),
        (p["l4"], None, None),
    ]):
        gin = g
        for (dst_eff, w_col, cnt_src, cnt_dst, Wsum) in subs:
            g = _conv(g, src, dst, dst_eff, w_col, cnt_src, cnt_dst, Wsum, lp)
        if li > 0:
            g = g + gin
        if gn is not None:
            g = relu(_bn_plain(g, gn, bn_))
    out = g @ p["Wf"].T + p["bf"]
    n = audio_mask.shape[0]
    a_idx = jnp.nonzero(audio_mask, size=n // 2)[0]
    v_idx = jnp.nonzero(~audio_mask, size=n // 2)[0]
    audio_out = jnp.take(gf, a_idx, axis=0) @ p["Wfa"].T + p["bfa"]
    video_out = jnp.take(gf, v_idx, axis=0) @ p["Wfv"].T + p["bfv"]
    return out, audio_out, video_out


# speedup vs baseline: 1.9228x; 1.9228x over previous
import functools

import jax
import jax.numpy as jnp
from jax import lax
from jax.experimental import pallas as pl
from jax.experimental.pallas import tpu as pltpu
from jax.experimental.pallas import tpu_sc as plsc

N = 10000
E = 320000
KCH = 128
NC, NS = 2, 16
NW = NC * NS
CPW = 80
EPW = CPW * KCH
EPAD = EPW * NW
CPT = EPAD // (NS * KCH)
EPT = CPT * KCH
NPAD = 79 * 128
DUMP = N
EPS = 1e-5
BE = 2048
G = EPAD // BE
ROWS_PT = NPAD // NS
H = 64
NSEG = 5000
NPH = 5120
RPT_D = NPH // NS

_MESH = plsc.VectorSubcoreMesh(core_axis_name="c", subcore_axis_name="s")


def _pad_e(a, fill):
    return jnp.concatenate([a, jnp.full((EPAD - E,), fill, a.dtype)])


@functools.partial(
    pl.kernel,
    out_type=[
        jax.ShapeDtypeStruct((EPAD, 128), jnp.float32),
        jax.ShapeDtypeStruct((NC, 8, 128), jnp.float32),
    ],
    mesh=_MESH,
    scratch_types=[
        pltpu.VMEM((CPW, KCH), jnp.int32),
        pltpu.VMEM((CPW, KCH), jnp.int32),
        pltpu.VMEM((CPW, KCH), jnp.int32),
        pltpu.VMEM((KCH, 128), jnp.float32),
        pltpu.VMEM((KCH, 128), jnp.float32),
        pltpu.VMEM((KCH, 128), jnp.float32),
        pltpu.VMEM_SHARED((8, 128), jnp.float32),
        pltpu.SemaphoreType.DMA,
        pltpu.SemaphoreType.DMA,
    ],
)
def _sc_gather_diff(x_hbm, srcv_hbm, dstv_hbm, wsel_hbm, zeros8_hbm,
                    d_hbm, ct_hbm,
                    src_v, dst_v, wsel_v, rows_s, rows_d, prod, ct_sh,
                    sem1, sem2):
    c = lax.axis_index("c")
    s = lax.axis_index("s")
    wid = c * NS + s

    @pl.when(s == 0)
    def _():
        pltpu.sync_copy(zeros8_hbm, ct_sh)

    pltpu.sync_copy(srcv_hbm.at[pl.ds(wid * CPW, CPW)], src_v)
    pltpu.sync_copy(dstv_hbm.at[pl.ds(wid * CPW, CPW)], dst_v)
    pltpu.sync_copy(wsel_hbm.at[pl.ds(wid * CPW, CPW)], wsel_v)
    plsc.subcore_barrier()

    def chunk(j, carry):
        cp1 = pltpu.async_copy(x_hbm.at[src_v.at[j]], rows_s, sem1)
        cp2 = pltpu.async_copy(x_hbm.at[dst_v.at[j]], rows_d, sem2)
        cp1.wait()
        cp2.wait()

        def edge(e, cc2):
            for t in range(8):
                sl = pl.ds(t * 16, 16)
                dv = rows_s[e, sl] - rows_d[e, sl]
                rows_s[e, sl] = dv
                prod[e, sl] = dv * dv
            return cc2

        lax.fori_loop(0, KCH, edge, 0)
        pltpu.sync_copy(prod, ct_sh.at[wsel_v.at[j]], add=True)
        pltpu.sync_copy(rows_s, d_hbm.at[pl.ds(wid * EPW + j * KCH, KCH)])
        return carry

    lax.fori_loop(0, CPW, chunk, 0)
    plsc.subcore_barrier()

    def flush(cc):
        @pl.when((s == 0) & (c == cc))
        def _():
            pltpu.sync_copy(ct_sh, ct_hbm.at[cc])

    flush(0)
    flush(1)


@functools.partial(
    pl.kernel,
    out_type=jax.ShapeDtypeStruct((NC, 8, 128), jnp.float32),
    mesh=_MESH,
    scratch_types=[
        pltpu.VMEM((CPW, KCH), jnp.int32),
        pltpu.VMEM((CPW, KCH), jnp.int32),
        pltpu.VMEM((KCH, 128), jnp.float32),
        pltpu.VMEM((KCH, 128), jnp.float32),
        pltpu.VMEM((KCH, 128), jnp.float32),
        pltpu.VMEM_SHARED((8, 128), jnp.float32),
        pltpu.SemaphoreType.DMA,
    ],
)
def _sc_cross(v_hbm, u_hbm, dstv_hbm, wsel_hbm, zeros8_hbm,
              ct_hbm,
              dst_v, wsel_v, vbuf, ubuf, prod, ct_sh, sem1):
    c = lax.axis_index("c")
    s = lax.axis_index("s")
    wid = c * NS + s

    @pl.when(s == 0)
    def _():
        pltpu.sync_copy(zeros8_hbm, ct_sh)

    pltpu.sync_copy(dstv_hbm.at[pl.ds(wid * CPW, CPW)], dst_v)
    pltpu.sync_copy(wsel_hbm.at[pl.ds(wid * CPW, CPW)], wsel_v)
    plsc.subcore_barrier()

    def chunk(j, carry):
        cp = pltpu.async_copy(u_hbm.at[dst_v.at[j]], ubuf, sem1)
        pltpu.sync_copy(v_hbm.at[pl.ds(wid * EPW + j * KCH, KCH)], vbuf)
        cp.wait()

        def edge(e, cc2):
            for t in range(8):
                sl = pl.ds(t * 16, 16)
                prod[e, sl] = ubuf[e, sl] * vbuf[e, sl]
            return cc2

        lax.fori_loop(0, KCH, edge, 0)
        pltpu.sync_copy(prod, ct_sh.at[wsel_v.at[j]], add=True)
        return carry

    lax.fori_loop(0, CPW, chunk, 0)
    plsc.subcore_barrier()

    def flush(cc):
        @pl.when((s == 0) & (c == cc))
        def _():
            pltpu.sync_copy(ct_sh, ct_hbm.at[cc])

    flush(0)
    flush(1)


@functools.partial(
    pl.kernel,
    out_type=jax.ShapeDtypeStruct((2, NC, NPH, 128), jnp.float32),
    mesh=_MESH,
    scratch_types=[
        pltpu.VMEM((CPW, KCH), jnp.int32),
        pltpu.VMEM((CPW, KCH), jnp.int32),
        pltpu.VMEM((KCH, 128), jnp.float32),
        pltpu.VMEM((KCH, 128), jnp.float32),
        pltpu.VMEM((2, 128), jnp.float32),
        pltpu.VMEM_SHARED((NPH, 128), jnp.float32),
        pltpu.SemaphoreType.DMA,
    ],
)
def _sc_final_edge(v_hbm, u_hbm, dstv_hbm, deff0_hbm, deff1_hbm, ab_hbm,
                   zeros_hbm,
                   s_hbm,
                   dst_v, deff_v, vbuf, ubuf, ab_v, s_sh, sem1):
    c = lax.axis_index("c")
    s = lax.axis_index("s")
    wid = c * NS + s
    pltpu.sync_copy(dstv_hbm.at[pl.ds(wid * CPW, CPW)], dst_v)
    pltpu.sync_copy(ab_hbm, ab_v)
    a_t = [ab_v[0, pl.ds(t * 16, 16)] for t in range(8)]
    b_t = [ab_v[1, pl.ds(t * 16, 16)] for t in range(8)]

    for r, deff_hbm in ((0, deff0_hbm), (1, deff1_hbm)):
        pltpu.sync_copy(zeros_hbm.at[pl.ds(s * RPT_D, RPT_D)],
                        s_sh.at[pl.ds(s * RPT_D, RPT_D)])
        pltpu.sync_copy(deff_hbm.at[pl.ds(wid * CPW, CPW)], deff_v)
        plsc.subcore_barrier()

        def chunk(j, carry):
            cp = pltpu.async_copy(u_hbm.at[dst_v.at[j]], ubuf, sem1)
            pltpu.sync_copy(v_hbm.at[pl.ds(wid * EPW + j * KCH, KCH)], vbuf)
            cp.wait()

            def edge(e, cc2):
                for t in range(8):
                    sl = pl.ds(t * 16, 16)
                    h = ubuf[e, sl] + vbuf[e, sl]
                    vbuf[e, sl] = jnp.maximum(h * a_t[t] + b_t[t], 0.0)
                return cc2

            lax.fori_loop(0, KCH, edge, 0)
            pltpu.sync_copy(vbuf, s_sh.at[deff_v.at[j]], add=True)
            return carry

        lax.fori_loop(0, CPW, chunk, 0)
        plsc.subcore_barrier()

        def flush(cc):
            @pl.when(c == cc)
            def _():
                pltpu.sync_copy(s_sh.at[pl.ds(s * RPT_D, RPT_D)],
                                s_hbm.at[r, cc, pl.ds(s * RPT_D, RPT_D)])

        flush(0)
        flush(1)
        plsc.subcore_barrier()


@functools.partial(
    pl.kernel,
    out_type=[
        jax.ShapeDtypeStruct((NC, NPAD, 16), jnp.float32),
        jax.ShapeDtypeStruct((NC, NPAD, 16), jnp.float32),
    ],
    mesh=_MESH,
    scratch_types=[
        pltpu.VMEM((CPW, KCH), jnp.int32),
        pltpu.VMEM((CPW, KCH), jnp.int32),
        pltpu.VMEM((KCH, 16), jnp.float32),
        pltpu.VMEM_SHARED((NPAD, 16), jnp.float32),
        pltpu.VMEM_SHARED((NPAD, 16), jnp.float32),
    ],
)
def _sc_counts(seff_hbm, deff_hbm, ones_hbm, zeros16_hbm, cs_hbm, cd_hbm,
               seff_v, deff_v, ones_v, cs_sh, cd_sh):
    c = lax.axis_index("c")
    s = lax.axis_index("s")
    wid = c * NS + s
    pltpu.sync_copy(zeros16_hbm.at[pl.ds(s * ROWS_PT, ROWS_PT)],
                    cs_sh.at[pl.ds(s * ROWS_PT, ROWS_PT)])
    pltpu.sync_copy(zeros16_hbm.at[pl.ds(s * ROWS_PT, ROWS_PT)],
                    cd_sh.at[pl.ds(s * ROWS_PT, ROWS_PT)])
    pltpu.sync_copy(ones_hbm, ones_v)
    pltpu.sync_copy(seff_hbm.at[pl.ds(wid * CPW, CPW)], seff_v)
    pltpu.sync_copy(deff_hbm.at[pl.ds(wid * CPW, CPW)], deff_v)
    plsc.subcore_barrier()

    def chunk(j, carry):
        pltpu.sync_copy(ones_v, cs_sh.at[seff_v.at[j]], add=True)
        pltpu.sync_copy(ones_v, cd_sh.at[deff_v.at[j]], add=True)
        return carry

    lax.fori_loop(0, CPW, chunk, 0)
    plsc.subcore_barrier()

    def flush(cc):
        @pl.when(c == cc)
        def _():
            pltpu.sync_copy(cs_sh.at[pl.ds(s * ROWS_PT, ROWS_PT)],
                            cs_hbm.at[cc, pl.ds(s * ROWS_PT, ROWS_PT)])
            pltpu.sync_copy(cd_sh.at[pl.ds(s * ROWS_PT, ROWS_PT)],
                            cd_hbm.at[cc, pl.ds(s * ROWS_PT, ROWS_PT)])

    flush(0)
    flush(1)


def _mm_stats_body(d_ref, w_ref, gam_ref, del_ref, W_ref,
                   v_ref, st_ref, acc_ref):
    i = pl.program_id(0)

    @pl.when(i == 0)
    def _init():
        acc_ref[...] = jnp.zeros_like(acc_ref)

    n = gam_ref[...] * d_ref[...] + del_ref[...]
    v = jnp.dot(jax.nn.relu(n), W_ref[...], preferred_element_type=jnp.float32)
    v_ref[...] = v
    w = (w_ref[...] < N).astype(jnp.float32)
    wv = v * w
    acc_ref[0:1, :] += jnp.sum(wv, axis=0, keepdims=True)
    acc_ref[1:2, :] += jnp.sum(wv * v, axis=0, keepdims=True)

    @pl.when(i == G - 1)
    def _fin():
        st_ref[...] = acc_ref[...]


def _mm_stats(d, w_col, gamma, delta, W1bT):
    return pl.pallas_call(
        _mm_stats_body,
        grid=(G,),
        in_specs=[
            pl.BlockSpec((BE, 128), lambda i: (i, 0)),
            pl.BlockSpec((BE, 1), lambda i: (i, 0)),
            pl.BlockSpec((1, 128), lambda i: (0, 0)),
            pl.BlockSpec((1, 128), lambda i: (0, 0)),
            pl.BlockSpec((128, 128), lambda i: (0, 0)),
        ],
        out_specs=[
            pl.BlockSpec((BE, 128), lambda i: (i, 0)),
            pl.BlockSpec((8, 128), lambda i: (0, 0)),
        ],
        out_shape=[
            jax.ShapeDtypeStruct((EPAD, 128), jnp.float32),
            jax.ShapeDtypeStruct((8, 128), jnp.float32),
        ],
        scratch_shapes=[pltpu.VMEM((8, 128), jnp.float32)],
    )(d, w_col, gamma.reshape(1, 128), delta.reshape(1, 128), W1bT)


def _conv(x, srcv, dstv, deff0v, deff1v, wselv, w_col, cnt_src, cnt_dst,
          Wsum, zerosD, zeros8, p):
    relu = jax.nn.relu
    d, ct = _sc_gather_diff(x, srcv, dstv, wselv, zeros8)
    SD2 = ct[0, 0] + ct[1, 0]
    cd = cnt_dst[:, None]
    cs = cnt_src[:, None]
    sx = (cd * x).sum(0)
    sx2 = (cd * x * x).sum(0)
    sj = (cs * x).sum(0)
    m_xi = sx / Wsum
    v_xi = sx2 / Wsum - m_xi**2
    m_d = (sj - sx) / Wsum
    v_d = SD2 / Wsum - m_d**2
    alpha = p["g1"][:128] * jax.lax.rsqrt(v_xi + EPS)
    beta = p["b1"][:128] - m_xi * alpha
    gamma = p["g1"][128:] * jax.lax.rsqrt(v_d + EPS)
    delta = p["b1"][128:] - m_d * gamma
    u = relu(alpha * x + beta) @ p["W1"][:, :128].T
    vfull, st = _mm_stats(d, w_col, gamma, delta, p["W1"][:, 128:].T)
    Sv, Sv2 = st[0], st[1]
    cr = _sc_cross(vfull, u, dstv, wselv, zeros8)
    UV = cr[0, 0] + cr[1, 0]
    Sh = (cd * u).sum(0) + Sv
    Sh2 = (cd * u * u).sum(0) + 2 * UV + Sv2
    m2 = Sh / Wsum
    var2 = Sh2 / Wsum - m2**2
    A = p["g2"] * jax.lax.rsqrt(var2 + EPS)
    B = p["b2"] - m2 * A
    ab = jnp.stack([A, B])
    Sp = _sc_final_edge(vfull, u, dstv, deff0v, deff1v, ab, zerosD)
    S = jnp.concatenate([(Sp[0, 0] + Sp[0, 1])[:NSEG],
                         (Sp[1, 0] + Sp[1, 1])[:NSEG]], axis=0)
    agg = S / jnp.maximum(cnt_dst, 1.0)[:, None]
    return agg @ p["W2"].T


def _bn_plain(x, g, b):
    m = x.mean(axis=0)
    v = x.var(axis=0)
    return (x - m) * jax.lax.rsqrt(v + EPS) * g + b


def kernel(x, edge_index, edge_delta, edge_self, audio_mask, params):
    relu = jax.nn.relu
    src = _pad_e(edge_index[0], 0)
    dst = _pad_e(edge_index[1], 0)
    m1 = _pad_e(edge_delta < 1, False)
    m2 = _pad_e(((edge_delta >= 1) & (edge_delta < 4)) | (edge_self == 1), False)
    srcv = src.reshape(EPAD // KCH, KCH)
    dstv = dst.reshape(EPAD // KCH, KCH)
    zerosD = jnp.zeros((NPH, 128), jnp.float32)
    zeros8 = jnp.zeros((8, 128), jnp.float32)
    zeros16 = jnp.zeros((NPAD, 16), jnp.float32)
    ones16 = jnp.ones((KCH, 16), jnp.float32)
    subs = []
    for m in (m1, m2):
        dst_eff = jnp.where(m, dst, DUMP)
        src_eff = jnp.where(m, src, DUMP)
        deffv = dst_eff.reshape(EPAD // KCH, KCH)
        seffv = src_eff.reshape(EPAD // KCH, KCH)
        deff0v = jnp.where(m & (dst < NSEG), dst,
                           NSEG).reshape(EPAD // KCH, KCH)
        deff1v = jnp.where(m & (dst >= NSEG), dst - NSEG,
                           NSEG).reshape(EPAD // KCH, KCH)
        wselv = jnp.where(m, 0, 1).astype(jnp.int32).reshape(EPAD // KCH, KCH)
        csp, cdp = _sc_counts(seffv, deffv, ones16, zeros16)
        cnt_src = (csp[0] + csp[1])[:N, 0]
        cnt_dst = (cdp[0] + cdp[1])[:N, 0]
        Wsum = cnt_dst.sum()
        w_col = dst_eff[:, None]
        subs.append((deff0v, deff1v, wselv, w_col, cnt_src, cnt_dst, Wsum))
    p = params
    a = x[:, 0, :] @ p["W0a"].T + p["b0a"]
    v = x[:, 1, :] @ p["W0v"].T + p["b0v"]
    gf = jnp.where(audio_mask[:, None], a, v)
    gf = relu(_bn_plain(gf, p["g0"], p["b0"]))
    g = gf
    for li, (lp, gn, bn_) in enumerate([
        (p["l1"], p["gb1"], p["bb1"]),
        (p["l2"], p["gb2"], p["bb2"]),
        (p["l3"], p["gb3"], p["bb3"]),
        (p["l4"], None, None),
    ]):
        gin = g
        for (deff0v, deff1v, wselv, w_col, cnt_src, cnt_dst, Wsum) in subs:
            g = _conv(g, srcv, dstv, deff0v, deff1v, wselv, w_col, cnt_src,
                      cnt_dst, Wsum, zerosD, zeros8, lp)
        if li > 0:
            g = g + gin
        if gn is not None:
            g = relu(_bn_plain(g, gn, bn_))
    out = g @ p["Wf"].T + p["bf"]
    n = audio_mask.shape[0]
    a_idx = jnp.nonzero(audio_mask, size=n // 2)[0]
    v_idx = jnp.nonzero(~audio_mask, size=n // 2)[0]
    audio_out = jnp.take(gf, a_idx, axis=0) @ p["Wfa"].T + p["bfa"]
    video_out = jnp.take(gf, v_idx, axis=0) @ p["Wfv"].T + p["bfv"]
    return out, audio_out, video_out


# speedup vs baseline: 7.1306x; 3.7083x over previous
import functools

import jax
import jax.numpy as jnp
from jax import lax
from jax.experimental import pallas as pl
from jax.experimental.pallas import tpu as pltpu
from jax.experimental.pallas import tpu_sc as plsc

N = 10000
E = 320000
KCH = 128
NC, NS = 2, 16
NW = NC * NS
CPW = 80
EPW = CPW * KCH
EPAD = EPW * NW
CPT = EPAD // (NS * KCH)
EPT = CPT * KCH
NPAD = 79 * 128
NZ = N
DUMP = N
EPS = 1e-5
BE = 2048
G = EPAD // BE
ROWS_PT = NPAD // NS
NSEG = 5000
NPH = 5120
RPT_D = NPH // NS

_MESH = plsc.VectorSubcoreMesh(core_axis_name="c", subcore_axis_name="s")


def _pad_e(a, fill):
    return jnp.concatenate([a, jnp.full((EPAD - E,), fill, a.dtype)])


@functools.partial(
    pl.kernel,
    out_type=[
        jax.ShapeDtypeStruct((EPAD, 128), jnp.float32),
        jax.ShapeDtypeStruct((NW, 128), jnp.float32),
    ],
    mesh=_MESH,
    scratch_types=[
        pltpu.VMEM((CPW, KCH), jnp.int32),
        pltpu.VMEM((CPW, KCH), jnp.int32),
        pltpu.VMEM((16,), jnp.int32),
        pltpu.VMEM((KCH, 128), jnp.float32),
        pltpu.VMEM((KCH, 128), jnp.float32),
        pltpu.VMEM((128,), jnp.float32),
        pltpu.SemaphoreType.DMA,
        pltpu.SemaphoreType.DMA,
    ],
)
def _sc_gather_diff(xz_hbm, srcz_hbm, dstz_hbm, cnt_hbm,
                    d_hbm, sd2_hbm,
                    src_v, dst_v, cnt_v, rows_s, rows_d, accb, sem1, sem2):
    c = lax.axis_index("c")
    s = lax.axis_index("s")
    wid = c * NS + s
    pltpu.sync_copy(srcz_hbm.at[pl.ds(wid * CPW, CPW)], src_v)
    pltpu.sync_copy(dstz_hbm.at[pl.ds(wid * CPW, CPW)], dst_v)
    pltpu.sync_copy(cnt_hbm.at[wid], cnt_v)
    nch = cnt_v[...][0]
    zero = jnp.zeros((16,), jnp.float32)

    def chunk(j, acc):
        cp1 = pltpu.async_copy(xz_hbm.at[src_v.at[j]], rows_s, sem1)
        cp2 = pltpu.async_copy(xz_hbm.at[dst_v.at[j]], rows_d, sem2)
        cp1.wait()
        cp2.wait()

        def edge(e, a2):
            out = []
            for t in range(8):
                sl = pl.ds(t * 16, 16)
                dv = rows_s[e, sl] - rows_d[e, sl]
                rows_s[e, sl] = dv
                out.append(a2[t] + dv * dv)
            return tuple(out)

        acc = lax.fori_loop(0, KCH, edge, acc)
        pltpu.sync_copy(rows_s, d_hbm.at[pl.ds(wid * EPW + j * KCH, KCH)])
        return acc

    acc = lax.fori_loop(0, nch, chunk, (zero,) * 8)
    for t in range(8):
        accb[pl.ds(t * 16, 16)] = acc[t]
    pltpu.sync_copy(accb, sd2_hbm.at[wid])


@functools.partial(
    pl.kernel,
    out_type=jax.ShapeDtypeStruct((NW, 128), jnp.float32),
    mesh=_MESH,
    scratch_types=[
        pltpu.VMEM((CPW, KCH), jnp.int32),
        pltpu.VMEM((16,), jnp.int32),
        pltpu.VMEM((KCH, 128), jnp.float32),
        pltpu.VMEM((KCH, 128), jnp.float32),
        pltpu.VMEM((128,), jnp.float32),
        pltpu.SemaphoreType.DMA,
    ],
)
def _sc_cross(v_hbm, uz_hbm, dstz_hbm, cnt_hbm,
              uv_hbm,
              dst_v, cnt_v, vbuf, ubuf, accb, sem1):
    c = lax.axis_index("c")
    s = lax.axis_index("s")
    wid = c * NS + s
    pltpu.sync_copy(dstz_hbm.at[pl.ds(wid * CPW, CPW)], dst_v)
    pltpu.sync_copy(cnt_hbm.at[wid], cnt_v)
    nch = cnt_v[...][0]
    zero = jnp.zeros((16,), jnp.float32)

    def chunk(j, acc):
        cp = pltpu.async_copy(uz_hbm.at[dst_v.at[j]], ubuf, sem1)
        pltpu.sync_copy(v_hbm.at[pl.ds(wid * EPW + j * KCH, KCH)], vbuf)
        cp.wait()

        def edge(e, a2):
            out = []
            for t in range(8):
                sl = pl.ds(t * 16, 16)
                out.append(a2[t] + ubuf[e, sl] * vbuf[e, sl])
            return tuple(out)

        return lax.fori_loop(0, KCH, edge, acc)

    acc = lax.fori_loop(0, nch, chunk, (zero,) * 8)
    for t in range(8):
        accb[pl.ds(t * 16, 16)] = acc[t]
    pltpu.sync_copy(accb, uv_hbm.at[wid])


@functools.partial(
    pl.kernel,
    out_type=jax.ShapeDtypeStruct((NC, NPH, 128), jnp.float32),
    mesh=_MESH,
    scratch_types=[
        pltpu.VMEM((CPT, KCH), jnp.int32),
        pltpu.VMEM((CPT, KCH), jnp.int32),
        pltpu.VMEM((2, 16), jnp.int32),
        pltpu.VMEM((KCH, 128), jnp.float32),
        pltpu.VMEM((KCH, 128), jnp.float32),
        pltpu.VMEM((2, 128), jnp.float32),
        pltpu.VMEM_SHARED((NPH, 128), jnp.float32),
        pltpu.SemaphoreType.DMA,
    ],
)
def _sc_final_edge(v_hbm, u_hbm, dstv_hbm, deffc0_hbm, deffc1_hbm, ab_hbm,
                   cnt_hbm, zeros_hbm,
                   s_hbm,
                   dst_v, deff_v, cnt_v, vbuf, ubuf, ab_v, s_sh, sem1):
    c = lax.axis_index("c")
    s = lax.axis_index("s")
    pltpu.sync_copy(zeros_hbm.at[pl.ds(s * RPT_D, RPT_D)],
                    s_sh.at[pl.ds(s * RPT_D, RPT_D)])
    pltpu.sync_copy(dstv_hbm.at[pl.ds(s * CPT, CPT)], dst_v)
    pltpu.sync_copy(cnt_hbm.at[pl.ds(2 * s, 2)], cnt_v)
    pltpu.sync_copy(ab_hbm, ab_v)

    def stage(cc, deff_hbm):
        @pl.when(c == cc)
        def _():
            pltpu.sync_copy(deff_hbm.at[pl.ds(s * CPT, CPT)], deff_v)

    stage(0, deffc0_hbm)
    stage(1, deffc1_hbm)
    plsc.subcore_barrier()
    a_t = [ab_v[0, pl.ds(t * 16, 16)] for t in range(8)]
    b_t = [ab_v[1, pl.ds(t * 16, 16)] for t in range(8)]
    n0 = cnt_v[0][0]
    n1 = cnt_v[1][0]

    for off, n_r in ((0, n0), (CPW, n1)):
        def chunk(i, carry):
            j = off + i
            cp = pltpu.async_copy(u_hbm.at[dst_v.at[j]], ubuf, sem1)
            pltpu.sync_copy(v_hbm.at[pl.ds(s * EPT + j * KCH, KCH)], vbuf)
            cp.wait()

            def edge(e, cc2):
                for t in range(8):
                    sl = pl.ds(t * 16, 16)
                    h = ubuf[e, sl] + vbuf[e, sl]
                    vbuf[e, sl] = jnp.maximum(h * a_t[t] + b_t[t], 0.0)
                return cc2

            lax.fori_loop(0, KCH, edge, 0)
            pltpu.sync_copy(vbuf, s_sh.at[deff_v.at[j]], add=True)
            return carry

        lax.fori_loop(0, n_r, chunk, 0)

    plsc.subcore_barrier()

    def flush(cc):
        @pl.when(c == cc)
        def _():
            pltpu.sync_copy(s_sh.at[pl.ds(s * RPT_D, RPT_D)],
                            s_hbm.at[cc, pl.ds(s * RPT_D, RPT_D)])

    flush(0)
    flush(1)


@functools.partial(
    pl.kernel,
    out_type=[
        jax.ShapeDtypeStruct((NC, NPAD, 16), jnp.float32),
        jax.ShapeDtypeStruct((NC, NPAD, 16), jnp.float32),
    ],
    mesh=_MESH,
    scratch_types=[
        pltpu.VMEM((CPW, KCH), jnp.int32),
        pltpu.VMEM((CPW, KCH), jnp.int32),
        pltpu.VMEM((16,), jnp.int32),
        pltpu.VMEM((KCH, 16), jnp.float32),
        pltpu.VMEM_SHARED((NPAD, 16), jnp.float32),
        pltpu.VMEM_SHARED((NPAD, 16), jnp.float32),
    ],
)
def _sc_counts(seff_hbm, deff_hbm, ones_hbm, zeros16_hbm, cnt_hbm,
               cs_hbm, cd_hbm,
               seff_v, deff_v, cnt_v, ones_v, cs_sh, cd_sh):
    c = lax.axis_index("c")
    s = lax.axis_index("s")
    wid = c * NS + s
    pltpu.sync_copy(zeros16_hbm.at[pl.ds(s * ROWS_PT, ROWS_PT)],
                    cs_sh.at[pl.ds(s * ROWS_PT, ROWS_PT)])
    pltpu.sync_copy(zeros16_hbm.at[pl.ds(s * ROWS_PT, ROWS_PT)],
                    cd_sh.at[pl.ds(s * ROWS_PT, ROWS_PT)])
    pltpu.sync_copy(ones_hbm, ones_v)
    pltpu.sync_copy(seff_hbm.at[pl.ds(wid * CPW, CPW)], seff_v)
    pltpu.sync_copy(deff_hbm.at[pl.ds(wid * CPW, CPW)], deff_v)
    pltpu.sync_copy(cnt_hbm.at[wid], cnt_v)
    nch = cnt_v[...][0]
    plsc.subcore_barrier()

    def chunk(j, carry):
        pltpu.sync_copy(ones_v, cs_sh.at[seff_v.at[j]], add=True)
        pltpu.sync_copy(ones_v, cd_sh.at[deff_v.at[j]], add=True)
        return carry

    lax.fori_loop(0, nch, chunk, 0)
    plsc.subcore_barrier()

    def flush(cc):
        @pl.when(c == cc)
        def _():
            pltpu.sync_copy(cs_sh.at[pl.ds(s * ROWS_PT, ROWS_PT)],
                            cs_hbm.at[cc, pl.ds(s * ROWS_PT, ROWS_PT)])
            pltpu.sync_copy(cd_sh.at[pl.ds(s * ROWS_PT, ROWS_PT)],
                            cd_hbm.at[cc, pl.ds(s * ROWS_PT, ROWS_PT)])

    flush(0)
    flush(1)


def _mm_stats_body(d_ref, w_ref, gam_ref, del_ref, W_ref,
                   v_ref, st_ref, acc_ref):
    i = pl.program_id(0)

    @pl.when(i == 0)
    def _init():
        acc_ref[...] = jnp.zeros_like(acc_ref)

    n = gam_ref[...] * d_ref[...] + del_ref[...]
    v = jnp.dot(jax.nn.relu(n), W_ref[...], preferred_element_type=jnp.float32)
    v_ref[...] = v
    w = w_ref[...] < N
    wv = jnp.where(w, v, 0.0)
    acc_ref[0:1, :] += jnp.sum(wv, axis=0, keepdims=True)
    acc_ref[1:2, :] += jnp.sum(wv * wv, axis=0, keepdims=True)

    @pl.when(i == G - 1)
    def _fin():
        st_ref[...] = acc_ref[...]


def _mm_stats(d, w_col, gamma, delta, W1bT):
    return pl.pallas_call(
        _mm_stats_body,
        grid=(G,),
        in_specs=[
            pl.BlockSpec((BE, 128), lambda i: (i, 0)),
            pl.BlockSpec((BE, 1), lambda i: (i, 0)),
            pl.BlockSpec((1, 128), lambda i: (0, 0)),
            pl.BlockSpec((1, 128), lambda i: (0, 0)),
            pl.BlockSpec((128, 128), lambda i: (0, 0)),
        ],
        out_specs=[
            pl.BlockSpec((BE, 128), lambda i: (i, 0)),
            pl.BlockSpec((8, 128), lambda i: (0, 0)),
        ],
        out_shape=[
            jax.ShapeDtypeStruct((EPAD, 128), jnp.float32),
            jax.ShapeDtypeStruct((8, 128), jnp.float32),
        ],
        scratch_shapes=[pltpu.VMEM((8, 128), jnp.float32)],
    )(d, w_col, gamma.reshape(1, 128), delta.reshape(1, 128), W1bT)


def _conv(x, sub, zerosD, p):
    relu = jax.nn.relu
    (srczv, dstzv, dstpv, deffc0v, deffc1v, w_col, cnt32b, cnt_src, cnt_dst,
     Wsum) = sub
    zrows = jnp.zeros((8, 128), jnp.float32)
    xz = jnp.concatenate([x, zrows], axis=0)
    d, sd2p = _sc_gather_diff(xz, srczv, dstzv, cnt32b)
    SD2 = sd2p.sum(0)
    cd = cnt_dst[:, None]
    cs = cnt_src[:, None]
    sx = (cd * x).sum(0)
    sx2 = (cd * x * x).sum(0)
    sj = (cs * x).sum(0)
    m_xi = sx / Wsum
    v_xi = sx2 / Wsum - m_xi**2
    m_d = (sj - sx) / Wsum
    v_d = SD2 / Wsum - m_d**2
    alpha = p["g1"][:128] * jax.lax.rsqrt(v_xi + EPS)
    beta = p["b1"][:128] - m_xi * alpha
    gamma = p["g1"][128:] * jax.lax.rsqrt(v_d + EPS)
    delta = p["b1"][128:] - m_d * gamma
    u = relu(alpha * x + beta) @ p["W1"][:, :128].T
    vfull, st = _mm_stats(d, w_col, gamma, delta, p["W1"][:, 128:].T)
    Sv, Sv2 = st[0], st[1]
    uz = jnp.concatenate([u, zrows], axis=0)
    crp = _sc_cross(vfull, uz, dstzv, cnt32b)
    UV = crp.sum(0)
    Sh = (cd * u).sum(0) + Sv
    Sh2 = (cd * u * u).sum(0) + 2 * UV + Sv2
    m2 = Sh / Wsum
    var2 = Sh2 / Wsum - m2**2
    A = p["g2"] * jax.lax.rsqrt(var2 + EPS)
    B = p["b2"] - m2 * A
    ab = jnp.stack([A, B])
    Sp = _sc_final_edge(vfull, u, dstpv, deffc0v, deffc1v, ab, cnt32b,
                        zerosD)
    S = jnp.concatenate([Sp[0][:NSEG], Sp[1][:NSEG]], axis=0)
    agg = S / jnp.maximum(cnt_dst, 1.0)[:, None]
    return agg @ p["W2"].T


def _bn_plain(x, g, b):
    m = x.mean(axis=0)
    v = x.var(axis=0)
    return (x - m) * jax.lax.rsqrt(v + EPS) * g + b


def kernel(x, edge_index, edge_delta, edge_self, audio_mask, params):
    relu = jax.nn.relu
    src = _pad_e(edge_index[0], 0)
    dst = _pad_e(edge_index[1], 0)
    m1 = _pad_e(edge_delta < 1, False)
    m2 = _pad_e(((edge_delta >= 1) & (edge_delta < 4)) | (edge_self == 1), False)
    zerosD = jnp.zeros((NPH, 128), jnp.float32)
    zeros16 = jnp.zeros((NPAD, 16), jnp.float32)
    ones16 = jnp.ones((KCH, 16), jnp.float32)
    sidx = jnp.arange(EPAD, dtype=jnp.int32)
    tt = sidx // EPW
    rem = sidx % EPW
    ci = ((rem // KCH) * NW + tt) * KCH + rem % KCH
    t32 = jnp.arange(NW, dtype=jnp.int32)
    subs = []
    for m in (m1, m2):
        nreal = m.sum().astype(jnp.int32)
        CH = (nreal + KCH - 1) // KCH
        cnt32 = jnp.maximum((CH - t32 + NW - 1) // NW, 0).astype(jnp.int32)
        cnt32b = jnp.tile(cnt32[:, None], (1, 16))
        cm = jnp.cumsum(m.astype(jnp.int32))
        cmn = jnp.cumsum((~m).astype(jnp.int32))
        pos = jnp.where(m, cm - 1, nreal + cmn - 1)
        order = jnp.zeros((EPAD,), jnp.int32).at[pos].set(sidx)
        gidx = jnp.take(order, ci)
        srcp = jnp.take(src, gidx)
        dstp = jnp.take(dst, gidx)
        mp = jnp.take(m, gidx)
        srczv = jnp.where(mp, srcp, NZ).reshape(EPAD // KCH, KCH)
        dstzv = jnp.where(mp, dstp, NZ).reshape(EPAD // KCH, KCH)
        dstpv = dstp.reshape(EPAD // KCH, KCH)
        deffc0v = jnp.where(mp & (dstp < NSEG), dstp,
                            NSEG).reshape(EPAD // KCH, KCH)
        deffc1v = jnp.where(mp & (dstp >= NSEG), dstp - NSEG,
                            NSEG).reshape(EPAD // KCH, KCH)
        deff = jnp.where(mp, dstp, DUMP)
        seffv = jnp.where(mp, srcp, DUMP).reshape(EPAD // KCH, KCH)
        csp, cdp = _sc_counts(seffv, deff.reshape(EPAD // KCH, KCH), ones16,
                              zeros16, cnt32b)
        cnt_src = (csp[0] + csp[1])[:N, 0]
        cnt_dst = (cdp[0] + cdp[1])[:N, 0]
        Wsum = cnt_dst.sum()
        w_col = deff[:, None]
        subs.append((srczv, dstzv, dstpv, deffc0v, deffc1v, w_col, cnt32b,
                     cnt_src, cnt_dst, Wsum))
    p = params
    a = x[:, 0, :] @ p["W0a"].T + p["b0a"]
    v = x[:, 1, :] @ p["W0v"].T + p["b0v"]
    gf = jnp.where(audio_mask[:, None], a, v)
    gf = relu(_bn_plain(gf, p["g0"], p["b0"]))
    g = gf
    for li, (lp, gn, bn_) in enumerate([
        (p["l1"], p["gb1"], p["bb1"]),
        (p["l2"], p["gb2"], p["bb2"]),
        (p["l3"], p["gb3"], p["bb3"]),
        (p["l4"], None, None),
    ]):
        gin = g
        for sub in subs:
            g = _conv(g, sub, zerosD, lp)
        if li > 0:
            g = g + gin
        if gn is not None:
            g = relu(_bn_plain(g, gn, bn_))
    out = g @ p["Wf"].T + p["bf"]
    n = audio_mask.shape[0]
    a_idx = jnp.nonzero(audio_mask, size=n // 2)[0]
    v_idx = jnp.nonzero(~audio_mask, size=n // 2)[0]
    audio_out = jnp.take(gf, a_idx, axis=0) @ p["Wfa"].T + p["bfa"]
    video_out = jnp.take(gf, v_idx, axis=0) @ p["Wfv"].T + p["bfv"]
    return out, audio_out, video_out


# speedup vs baseline: 8.0884x; 1.1343x over previous
import functools

import jax
import jax.numpy as jnp
from jax import lax
from jax.experimental import pallas as pl
from jax.experimental.pallas import tpu as pltpu
from jax.experimental.pallas import tpu_sc as plsc

N = 10000
E = 320000
KCH = 128
NC, NS = 2, 16
NW = NC * NS
CPW = 80
EPW = CPW * KCH
EPAD = EPW * NW
CPT = EPAD // (NS * KCH)
EPT = CPT * KCH
NPAD = 79 * 128
NZ = N
DUMP = N
EPS = 1e-5
BE = 2048
G = EPAD // BE
ROWS_PT = NPAD // NS
NSEG = 5000
NPH = 5120
RPT_D = NPH // NS

_MESH = plsc.VectorSubcoreMesh(core_axis_name="c", subcore_axis_name="s")


def _pad_e(a, fill):
    return jnp.concatenate([a, jnp.full((EPAD - E,), fill, a.dtype)])


@functools.partial(
    pl.kernel,
    out_type=[
        jax.ShapeDtypeStruct((EPAD, 128), jnp.float32),
        jax.ShapeDtypeStruct((NW, 128), jnp.float32),
    ],
    mesh=_MESH,
    scratch_types=[
        pltpu.VMEM((CPW, KCH), jnp.int32),
        pltpu.VMEM((CPW, KCH), jnp.int32),
        pltpu.VMEM((16,), jnp.int32),
        pltpu.VMEM((KCH, 128), jnp.float32),
        pltpu.VMEM((KCH, 128), jnp.float32),
        pltpu.VMEM((KCH, 128), jnp.float32),
        pltpu.VMEM((KCH, 128), jnp.float32),
        pltpu.VMEM((128,), jnp.float32),
        pltpu.SemaphoreType.DMA,
        pltpu.SemaphoreType.DMA,
        pltpu.SemaphoreType.DMA,
        pltpu.SemaphoreType.DMA,
    ],
)
def _sc_gather_diff(xz_hbm, srcz_hbm, dstz_hbm, cnt_hbm,
                    d_hbm, sd2_hbm,
                    src_v, dst_v, cnt_v, rs0, rd0, rs1, rd1, accb,
                    ss0, sd0, ss1, sd1):
    c = lax.axis_index("c")
    s = lax.axis_index("s")
    wid = c * NS + s
    pltpu.sync_copy(srcz_hbm.at[pl.ds(wid * CPW, CPW)], src_v)
    pltpu.sync_copy(dstz_hbm.at[pl.ds(wid * CPW, CPW)], dst_v)
    pltpu.sync_copy(cnt_hbm.at[wid], cnt_v)
    nch = cnt_v[...][0]
    zero = jnp.zeros((16,), jnp.float32)
    bufs = ((rs0, rd0, ss0, sd0), (rs1, rd1, ss1, sd1))

    for b in range(2):
        rs, rd, ss, sd = bufs[b]

        @pl.when(b < nch)
        def _(rs=rs, rd=rd, ss=ss, sd=sd, b=b):
            pltpu.async_copy(xz_hbm.at[src_v.at[b]], rs, ss)
            pltpu.async_copy(xz_hbm.at[dst_v.at[b]], rd, sd)

    def pair(jp, acc):
        for b in range(2):
            rs, rd, ss, sd = bufs[b]
            j = jp * 2 + b
            live = j < nch

            @pl.when(live)
            def _(rs=rs, rd=rd, ss=ss, sd=sd, j=j):
                pltpu.make_async_copy(xz_hbm.at[src_v.at[j]], rs, ss).wait()
                pltpu.make_async_copy(xz_hbm.at[dst_v.at[j]], rd, sd).wait()

            def edge(e, a2, rs=rs, rd=rd):
                out = []
                for t in range(8):
                    sl = pl.ds(t * 16, 16)
                    dv = rs[e, sl] - rd[e, sl]
                    rs[e, sl] = dv
                    out.append(a2[t] + dv * dv)
                return tuple(out)

            acc2 = lax.fori_loop(0, KCH, edge, acc)
            acc = tuple(jnp.where(live, a2, a1)
                        for a1, a2 in zip(acc, acc2))

            @pl.when(live)
            def _(rs=rs, j=j):
                pltpu.sync_copy(rs,
                                d_hbm.at[pl.ds(wid * EPW + j * KCH, KCH)])

            @pl.when(j + 2 < nch)
            def _(rs=rs, rd=rd, ss=ss, sd=sd, j=j):
                pltpu.async_copy(xz_hbm.at[src_v.at[j + 2]], rs, ss)
                pltpu.async_copy(xz_hbm.at[dst_v.at[j + 2]], rd, sd)
        return acc

    acc = lax.fori_loop(0, (nch + 1) // 2, pair, (zero,) * 8)
    for t in range(8):
        accb[pl.ds(t * 16, 16)] = acc[t]
    pltpu.sync_copy(accb, sd2_hbm.at[wid])


@functools.partial(
    pl.kernel,
    out_type=jax.ShapeDtypeStruct((NW, 128), jnp.float32),
    mesh=_MESH,
    scratch_types=[
        pltpu.VMEM((CPW, KCH), jnp.int32),
        pltpu.VMEM((16,), jnp.int32),
        pltpu.VMEM((KCH, 128), jnp.float32),
        pltpu.VMEM((KCH, 128), jnp.float32),
        pltpu.VMEM((KCH, 128), jnp.float32),
        pltpu.VMEM((KCH, 128), jnp.float32),
        pltpu.VMEM((128,), jnp.float32),
        pltpu.SemaphoreType.DMA,
        pltpu.SemaphoreType.DMA,
        pltpu.SemaphoreType.DMA,
        pltpu.SemaphoreType.DMA,
    ],
)
def _sc_cross(v_hbm, uz_hbm, dstz_hbm, cnt_hbm,
              uv_hbm,
              dst_v, cnt_v, vb0, ub0, vb1, ub1, accb, sv0, su0, sv1, su1):
    c = lax.axis_index("c")
    s = lax.axis_index("s")
    wid = c * NS + s
    pltpu.sync_copy(dstz_hbm.at[pl.ds(wid * CPW, CPW)], dst_v)
    pltpu.sync_copy(cnt_hbm.at[wid], cnt_v)
    nch = cnt_v[...][0]
    zero = jnp.zeros((16,), jnp.float32)
    bufs = ((vb0, ub0, sv0, su0), (vb1, ub1, sv1, su1))

    def issue(j, vb, ub, sv, su):
        pltpu.async_copy(uz_hbm.at[dst_v.at[j]], ub, su)
        pltpu.async_copy(v_hbm.at[pl.ds(wid * EPW + j * KCH, KCH)], vb, sv)

    for b in range(2):
        vb, ub, sv, su = bufs[b]

        @pl.when(b < nch)
        def _(vb=vb, ub=ub, sv=sv, su=su, b=b):
            issue(b, vb, ub, sv, su)

    def pair(jp, acc):
        for b in range(2):
            vb, ub, sv, su = bufs[b]
            j = jp * 2 + b
            live = j < nch

            @pl.when(live)
            def _(vb=vb, ub=ub, sv=sv, su=su, j=j):
                pltpu.make_async_copy(uz_hbm.at[dst_v.at[j]], ub, su).wait()
                pltpu.make_async_copy(
                    v_hbm.at[pl.ds(wid * EPW + j * KCH, KCH)], vb, sv).wait()

            def edge(e, a2, vb=vb, ub=ub):
                out = []
                for t in range(8):
                    sl = pl.ds(t * 16, 16)
                    out.append(a2[t] + ub[e, sl] * vb[e, sl])
                return tuple(out)

            acc2 = lax.fori_loop(0, KCH, edge, acc)
            acc = tuple(jnp.where(live, a2, a1)
                        for a1, a2 in zip(acc, acc2))

            @pl.when(j + 2 < nch)
            def _(vb=vb, ub=ub, sv=sv, su=su, j=j):
                issue(j + 2, vb, ub, sv, su)
        return acc

    acc = lax.fori_loop(0, (nch + 1) // 2, pair, (zero,) * 8)
    for t in range(8):
        accb[pl.ds(t * 16, 16)] = acc[t]
    pltpu.sync_copy(accb, uv_hbm.at[wid])


@functools.partial(
    pl.kernel,
    out_type=jax.ShapeDtypeStruct((NC, NPH, 128), jnp.float32),
    mesh=_MESH,
    scratch_types=[
        pltpu.VMEM((CPW, KCH), jnp.int32),
        pltpu.VMEM((CPW, KCH), jnp.int32),
        pltpu.VMEM((2, 16), jnp.int32),
        pltpu.VMEM((KCH, 128), jnp.float32),
        pltpu.VMEM((KCH, 128), jnp.float32),
        pltpu.VMEM((KCH, 128), jnp.float32),
        pltpu.VMEM((KCH, 128), jnp.float32),
        pltpu.VMEM((2, 128), jnp.float32),
        pltpu.VMEM_SHARED((NPH, 128), jnp.float32),
        pltpu.SemaphoreType.DMA,
        pltpu.SemaphoreType.DMA,
        pltpu.SemaphoreType.DMA,
        pltpu.SemaphoreType.DMA,
    ],
)
def _sc_final_edge(v_hbm, u_hbm, dstv_hbm, deffc0_hbm, deffc1_hbm, ab_hbm,
                   cnt_hbm, zeros_hbm,
                   s_hbm,
                   dst_v, deff_v, cnt_v, vb0, ub0, vb1, ub1, ab_v, s_sh,
                   sv0, su0, sv1, su1):
    c = lax.axis_index("c")
    s = lax.axis_index("s")
    pltpu.sync_copy(zeros_hbm.at[pl.ds(s * RPT_D, RPT_D)],
                    s_sh.at[pl.ds(s * RPT_D, RPT_D)])
    pltpu.sync_copy(cnt_hbm.at[pl.ds(2 * s, 2)], cnt_v)
    pltpu.sync_copy(ab_hbm, ab_v)
    plsc.subcore_barrier()
    a_t = [ab_v[0, pl.ds(t * 16, 16)] for t in range(8)]
    b_t = [ab_v[1, pl.ds(t * 16, 16)] for t in range(8)]
    n0 = cnt_v[0][0]
    n1 = cnt_v[1][0]
    bufs = ((vb0, ub0, sv0, su0), (vb1, ub1, sv1, su1))

    for off, n_r, deff_hbm in ((0, n0, deffc0_hbm), (CPW, n1, deffc1_hbm)):
        pltpu.sync_copy(dstv_hbm.at[pl.ds(s * CPT + off, CPW)], dst_v)

        def stage(cc, deff_hbm=deff_hbm, off=off):
            @pl.when(c == cc)
            def _():
                pltpu.sync_copy(deff_hbm.at[pl.ds(s * CPT + off, CPW)],
                                deff_v)

        stage(0)
        stage(1)

        def issue(jr, vb, ub, sv, su, off=off):
            pltpu.async_copy(u_hbm.at[dst_v.at[jr]], ub, su)
            pltpu.async_copy(
                v_hbm.at[pl.ds(s * EPT + (off + jr) * KCH, KCH)], vb, sv)

        for b in range(2):
            vb, ub, sv, su = bufs[b]

            @pl.when(b < n_r)
            def _(vb=vb, ub=ub, sv=sv, su=su, b=b, issue=issue):
                issue(b, vb, ub, sv, su)

        def pair(jp, carry, off=off, n_r=n_r, issue=issue):
            for b in range(2):
                vb, ub, sv, su = bufs[b]
                jr = jp * 2 + b
                live = jr < n_r

                @pl.when(live)
                def _(vb=vb, ub=ub, sv=sv, su=su, jr=jr):
                    pltpu.make_async_copy(u_hbm.at[dst_v.at[jr]], ub,
                                          su).wait()
                    pltpu.make_async_copy(
                        v_hbm.at[pl.ds(s * EPT + (off + jr) * KCH, KCH)],
                        vb, sv).wait()

                def edge(e, cc2, vb=vb, ub=ub):
                    for t in range(8):
                        sl = pl.ds(t * 16, 16)
                        h = ub[e, sl] + vb[e, sl]
                        vb[e, sl] = jnp.maximum(h * a_t[t] + b_t[t], 0.0)
                    return cc2

                lax.fori_loop(0, KCH, edge, 0)

                @pl.when(live)
                def _(vb=vb, jr=jr):
                    pltpu.sync_copy(vb, s_sh.at[deff_v.at[jr]], add=True)

                @pl.when(jr + 2 < n_r)
                def _(vb=vb, ub=ub, sv=sv, su=su, jr=jr):
                    issue(jr + 2, vb, ub, sv, su)
            return carry

        lax.fori_loop(0, (n_r + 1) // 2, pair, 0)

    plsc.subcore_barrier()

    def flush(cc):
        @pl.when(c == cc)
        def _():
            pltpu.sync_copy(s_sh.at[pl.ds(s * RPT_D, RPT_D)],
                            s_hbm.at[cc, pl.ds(s * RPT_D, RPT_D)])

    flush(0)
    flush(1)


@functools.partial(
    pl.kernel,
    out_type=[
        jax.ShapeDtypeStruct((NC, NPAD, 16), jnp.float32),
        jax.ShapeDtypeStruct((NC, NPAD, 16), jnp.float32),
    ],
    mesh=_MESH,
    scratch_types=[
        pltpu.VMEM((CPW, KCH), jnp.int32),
        pltpu.VMEM((CPW, KCH), jnp.int32),
        pltpu.VMEM((16,), jnp.int32),
        pltpu.VMEM((KCH, 16), jnp.float32),
        pltpu.VMEM_SHARED((NPAD, 16), jnp.float32),
        pltpu.VMEM_SHARED((NPAD, 16), jnp.float32),
    ],
)
def _sc_counts(seff_hbm, deff_hbm, ones_hbm, zeros16_hbm, cnt_hbm,
               cs_hbm, cd_hbm,
               seff_v, deff_v, cnt_v, ones_v, cs_sh, cd_sh):
    c = lax.axis_index("c")
    s = lax.axis_index("s")
    wid = c * NS + s
    pltpu.sync_copy(zeros16_hbm.at[pl.ds(s * ROWS_PT, ROWS_PT)],
                    cs_sh.at[pl.ds(s * ROWS_PT, ROWS_PT)])
    pltpu.sync_copy(zeros16_hbm.at[pl.ds(s * ROWS_PT, ROWS_PT)],
                    cd_sh.at[pl.ds(s * ROWS_PT, ROWS_PT)])
    pltpu.sync_copy(ones_hbm, ones_v)
    pltpu.sync_copy(seff_hbm.at[pl.ds(wid * CPW, CPW)], seff_v)
    pltpu.sync_copy(deff_hbm.at[pl.ds(wid * CPW, CPW)], deff_v)
    pltpu.sync_copy(cnt_hbm.at[wid], cnt_v)
    nch = cnt_v[...][0]
    plsc.subcore_barrier()

    def chunk(j, carry):
        pltpu.sync_copy(ones_v, cs_sh.at[seff_v.at[j]], add=True)
        pltpu.sync_copy(ones_v, cd_sh.at[deff_v.at[j]], add=True)
        return carry

    lax.fori_loop(0, nch, chunk, 0)
    plsc.subcore_barrier()

    def flush(cc):
        @pl.when(c == cc)
        def _():
            pltpu.sync_copy(cs_sh.at[pl.ds(s * ROWS_PT, ROWS_PT)],
                            cs_hbm.at[cc, pl.ds(s * ROWS_PT, ROWS_PT)])
            pltpu.sync_copy(cd_sh.at[pl.ds(s * ROWS_PT, ROWS_PT)],
                            cd_hbm.at[cc, pl.ds(s * ROWS_PT, ROWS_PT)])

    flush(0)
    flush(1)


def _mm_stats_body(d_ref, w_ref, gam_ref, del_ref, W_ref,
                   v_ref, st_ref, acc_ref):
    i = pl.program_id(0)

    @pl.when(i == 0)
    def _init():
        acc_ref[...] = jnp.zeros_like(acc_ref)

    n = gam_ref[...] * d_ref[...] + del_ref[...]
    v = jnp.dot(jax.nn.relu(n), W_ref[...], preferred_element_type=jnp.float32)
    v_ref[...] = v
    w = w_ref[...] < N
    wv = jnp.where(w, v, 0.0)
    acc_ref[0:1, :] += jnp.sum(wv, axis=0, keepdims=True)
    acc_ref[1:2, :] += jnp.sum(wv * wv, axis=0, keepdims=True)

    @pl.when(i == G - 1)
    def _fin():
        st_ref[...] = acc_ref[...]


def _mm_stats(d, w_col, gamma, delta, W1bT):
    return pl.pallas_call(
        _mm_stats_body,
        grid=(G,),
        in_specs=[
            pl.BlockSpec((BE, 128), lambda i: (i, 0)),
            pl.BlockSpec((BE, 1), lambda i: (i, 0)),
            pl.BlockSpec((1, 128), lambda i: (0, 0)),
            pl.BlockSpec((1, 128), lambda i: (0, 0)),
            pl.BlockSpec((128, 128), lambda i: (0, 0)),
        ],
        out_specs=[
            pl.BlockSpec((BE, 128), lambda i: (i, 0)),
            pl.BlockSpec((8, 128), lambda i: (0, 0)),
        ],
        out_shape=[
            jax.ShapeDtypeStruct((EPAD, 128), jnp.float32),
            jax.ShapeDtypeStruct((8, 128), jnp.float32),
        ],
        scratch_shapes=[pltpu.VMEM((8, 128), jnp.float32)],
    )(d, w_col, gamma.reshape(1, 128), delta.reshape(1, 128), W1bT)


def _conv(x, sub, zerosD, p):
    relu = jax.nn.relu
    (srczv, dstzv, dstpv, deffc0v, deffc1v, w_col, cnt32b, cnt_src, cnt_dst,
     Wsum) = sub
    zrows = jnp.zeros((8, 128), jnp.float32)
    xz = jnp.concatenate([x, zrows], axis=0)
    d, sd2p = _sc_gather_diff(xz, srczv, dstzv, cnt32b)
    SD2 = sd2p.sum(0)
    cd = cnt_dst[:, None]
    cs = cnt_src[:, None]
    sx = (cd * x).sum(0)
    sx2 = (cd * x * x).sum(0)
    sj = (cs * x).sum(0)
    m_xi = sx / Wsum
    v_xi = sx2 / Wsum - m_xi**2
    m_d = (sj - sx) / Wsum
    v_d = SD2 / Wsum - m_d**2
    alpha = p["g1"][:128] * jax.lax.rsqrt(v_xi + EPS)
    beta = p["b1"][:128] - m_xi * alpha
    gamma = p["g1"][128:] * jax.lax.rsqrt(v_d + EPS)
    delta = p["b1"][128:] - m_d * gamma
    u = relu(alpha * x + beta) @ p["W1"][:, :128].T
    vfull, st = _mm_stats(d, w_col, gamma, delta, p["W1"][:, 128:].T)
    Sv, Sv2 = st[0], st[1]
    uz = jnp.concatenate([u, zrows], axis=0)
    crp = _sc_cross(vfull, uz, dstzv, cnt32b)
    UV = crp.sum(0)
    Sh = (cd * u).sum(0) + Sv
    Sh2 = (cd * u * u).sum(0) + 2 * UV + Sv2
    m2 = Sh / Wsum
    var2 = Sh2 / Wsum - m2**2
    A = p["g2"] * jax.lax.rsqrt(var2 + EPS)
    B = p["b2"] - m2 * A
    ab = jnp.stack([A, B])
    Sp = _sc_final_edge(vfull, u, dstpv, deffc0v, deffc1v, ab, cnt32b,
                        zerosD)
    S = jnp.concatenate([Sp[0][:NSEG], Sp[1][:NSEG]], axis=0)
    agg = S / jnp.maximum(cnt_dst, 1.0)[:, None]
    return agg @ p["W2"].T


def _bn_plain(x, g, b):
    m = x.mean(axis=0)
    v = x.var(axis=0)
    return (x - m) * jax.lax.rsqrt(v + EPS) * g + b


def kernel(x, edge_index, edge_delta, edge_self, audio_mask, params):
    relu = jax.nn.relu
    src = _pad_e(edge_index[0], 0)
    dst = _pad_e(edge_index[1], 0)
    m1 = _pad_e(edge_delta < 1, False)
    m2 = _pad_e(((edge_delta >= 1) & (edge_delta < 4)) | (edge_self == 1), False)
    zerosD = jnp.zeros((NPH, 128), jnp.float32)
    zeros16 = jnp.zeros((NPAD, 16), jnp.float32)
    ones16 = jnp.ones((KCH, 16), jnp.float32)
    sidx = jnp.arange(EPAD, dtype=jnp.int32)
    tt = sidx // EPW
    rem = sidx % EPW
    ci = ((rem // KCH) * NW + tt) * KCH + rem % KCH
    t32 = jnp.arange(NW, dtype=jnp.int32)
    subs = []
    for m in (m1, m2):
        nreal = m.sum().astype(jnp.int32)
        CH = (nreal + KCH - 1) // KCH
        cnt32 = jnp.maximum((CH - t32 + NW - 1) // NW, 0).astype(jnp.int32)
        cnt32b = jnp.tile(cnt32[:, None], (1, 16))
        cm = jnp.cumsum(m.astype(jnp.int32))
        cmn = jnp.cumsum((~m).astype(jnp.int32))
        pos = jnp.where(m, cm - 1, nreal + cmn - 1)
        order = jnp.zeros((EPAD,), jnp.int32).at[pos].set(sidx)
        gidx = jnp.take(order, ci)
        srcp = jnp.take(src, gidx)
        dstp = jnp.take(dst, gidx)
        mp = jnp.take(m, gidx)
        srczv = jnp.where(mp, srcp, NZ).reshape(EPAD // KCH, KCH)
        dstzv = jnp.where(mp, dstp, NZ).reshape(EPAD // KCH, KCH)
        dstpv = dstp.reshape(EPAD // KCH, KCH)
        deffc0v = jnp.where(mp & (dstp < NSEG), dstp,
                            NSEG).reshape(EPAD // KCH, KCH)
        deffc1v = jnp.where(mp & (dstp >= NSEG), dstp - NSEG,
                            NSEG).reshape(EPAD // KCH, KCH)
        deff = jnp.where(mp, dstp, DUMP)
        seffv = jnp.where(mp, srcp, DUMP).reshape(EPAD // KCH, KCH)
        csp, cdp = _sc_counts(seffv, deff.reshape(EPAD // KCH, KCH), ones16,
                              zeros16, cnt32b)
        cnt_src = (csp[0] + csp[1])[:N, 0]
        cnt_dst = (cdp[0] + cdp[1])[:N, 0]
        Wsum = cnt_dst.sum()
        w_col = deff[:, None]
        subs.append((srczv, dstzv, dstpv, deffc0v, deffc1v, w_col, cnt32b,
                     cnt_src, cnt_dst, Wsum))
    p = params
    a = x[:, 0, :] @ p["W0a"].T + p["b0a"]
    v = x[:, 1, :] @ p["W0v"].T + p["b0v"]
    gf = jnp.where(audio_mask[:, None], a, v)
    gf = relu(_bn_plain(gf, p["g0"], p["b0"]))
    g = gf
    for li, (lp, gn, bn_) in enumerate([
        (p["l1"], p["gb1"], p["bb1"]),
        (p["l2"], p["gb2"], p["bb2"]),
        (p["l3"], p["gb3"], p["bb3"]),
        (p["l4"], None, None),
    ]):
        gin = g
        for sub in subs:
            g = _conv(g, sub, zerosD, lp)
        if li > 0:
            g = g + gin
        if gn is not None:
            g = relu(_bn_plain(g, gn, bn_))
    out = g @ p["Wf"].T + p["bf"]
    n = audio_mask.shape[0]
    a_idx = jnp.nonzero(audio_mask, size=n // 2)[0]
    v_idx = jnp.nonzero(~audio_mask, size=n // 2)[0]
    audio_out = jnp.take(gf, a_idx, axis=0) @ p["Wfa"].T + p["bfa"]
    video_out = jnp.take(gf, v_idx, axis=0) @ p["Wfv"].T + p["bfv"]
    return out, audio_out, video_out
